# baseline (speedup 1.0000x reference)
import functools

import jax
import jax.numpy as jnp
from jax import lax
from jax.experimental import pallas as pl
from jax.experimental.pallas import tpu as pltpu
from jax.experimental.pallas import tpu_sc as plsc

NUM_CORES = 2
NUM_SUBCORES = 16
NUM_WORKERS = NUM_CORES * NUM_SUBCORES
BATCH = 16384
EMBED_DIM = 64
B_PER_W = BATCH // NUM_WORKERS
CHUNK = 256
NCHUNK = B_PER_W // CHUNK
LANES = 16
NROWS = 1000000
TROWS = 500000
TCOLS = 128

CW = 512
NFULL = 1953
TAILROWS = 64
MAXT = 62

_mesh = plsc.VectorSubcoreMesh(core_axis_name="c", subcore_axis_name="s")

_DNUMS = lax.GatherDimensionNumbers(
    offset_dims=(), collapsed_slice_dims=(0,), start_index_map=(0,))


@functools.partial(
    pl.kernel,
    mesh=_mesh,
    compiler_params=pltpu.CompilerParams(use_tc_tiling_on_sc=True,
                                         needs_layout_passes=False),
    out_type=jax.ShapeDtypeStruct((TROWS, TCOLS), jnp.float32),
    scratch_types=[
        pltpu.VMEM((EMBED_DIM, CW), jnp.float32),
        pltpu.VMEM((EMBED_DIM, CW), jnp.float32),
        pltpu.VMEM((CW // 2, TCOLS), jnp.float32),
        pltpu.VMEM((CW // 2, TCOLS), jnp.float32),
        pltpu.SemaphoreType.DMA,
        pltpu.SemaphoreType.DMA,
        pltpu.SemaphoreType.DMA,
        pltpu.SemaphoreType.DMA,
    ],
)
def _k0(tt, tail2, out, pan0, pan1, ob0, ob1, si0, si1, so0, so1):
    wid = lax.axis_index("s") * NUM_CORES + lax.axis_index("c")
    lane = lax.iota(jnp.int32, LANES)

    def fire(c, pan, sem):
        pltpu.async_copy(tt.at[:, pl.ds(c * CW, CW)], pan, sem)

    def step(t, pan, ob, si, so, pan_nx, si_nx):
        c = wid + NUM_WORKERS * t

        @pl.when(c < NFULL)
        def _():
            cn = wid + NUM_WORKERS * (t + 1)

            @pl.when(cn < NFULL)
            def _():
                fire(cn, pan_nx, si_nx)

            pltpu.make_async_copy(tt.at[:, pl.ds(c * CW, CW)], pan, si).wait()

            @pl.when(t >= 2)
            def _():
                pltpu.make_async_copy(
                    ob, out.at[pl.ds(0, CW // 2), :], so).wait()

            @pl.loop(0, CW // 2, step=8)
            def _(q0):
                cb = jnp.full((LANES,), 0, jnp.int32) + 2 * q0
                for dq in range(8):
                    for h in range(2):
                        col = cb + (2 * dq + h)
                        for k in range(4):
                            v = plsc.load_gather(pan, [lane + LANES * k, col])
                            ob[q0 + dq,
                               pl.ds(h * EMBED_DIM + LANES * k, LANES)] = v

            pltpu.async_copy(ob, out.at[pl.ds(c * (CW // 2), CW // 2), :], so)

    fire(wid, pan0, si0)

    @pl.loop(0, MAXT, step=2)
    def _(t):
        step(t, pan0, ob0, si0, so0, pan1, si1)
        step(t + 1, pan1, ob1, si1, so1, pan0, si0)

    nc = (NFULL - wid + NUM_WORKERS - 1) // NUM_WORKERS

    @pl.when(nc >= 2)
    def _():
        pltpu.make_async_copy(ob1, out.at[pl.ds(0, CW // 2), :], so1).wait()

    @pl.when(nc >= 1)
    def _():
        pltpu.make_async_copy(ob0, out.at[pl.ds(0, CW // 2), :], so0).wait()

    @pl.when(wid == 0)
    def _():
        pltpu.sync_copy(tail2, ob0.at[pl.ds(0, TAILROWS // 2), :])
        pltpu.sync_copy(ob0.at[pl.ds(0, TAILROWS // 2), :],
                        out.at[pl.ds(NFULL * CW // 2, TAILROWS // 2), :])


@functools.partial(
    pl.kernel,
    mesh=_mesh,
    compiler_params=pltpu.CompilerParams(use_tc_tiling_on_sc=True),
    out_type=[
        jax.ShapeDtypeStruct((BATCH,), jnp.float32),
        jax.ShapeDtypeStruct((BATCH,), jnp.float32),
    ],
    scratch_types=[
        pltpu.VMEM((B_PER_W,), jnp.int32),
        pltpu.VMEM((B_PER_W,), jnp.int32),
        pltpu.VMEM((B_PER_W,), jnp.int32),
        pltpu.VMEM((CHUNK,), jnp.int32),
        pltpu.VMEM((CHUNK,), jnp.int32),
        pltpu.VMEM((CHUNK,), jnp.int32),
        pltpu.VMEM((CHUNK, TCOLS), jnp.float32),
        pltpu.VMEM((CHUNK, TCOLS), jnp.float32),
        pltpu.VMEM((CHUNK, TCOLS), jnp.float32),
        pltpu.VMEM((B_PER_W,), jnp.float32),
        pltpu.VMEM((B_PER_W,), jnp.float32),
        pltpu.SemaphoreType.DMA,
    ],
)
def _k1(u_tab, i_tab, uid, pid, nid, pos_out, neg_out,
        uidx, pidx, nidx, g_u, g_p, g_n,
        urows, prows, nrows, pos_v, neg_v, sem):
    wid = lax.axis_index("s") * NUM_CORES + lax.axis_index("c")
    base = wid * B_PER_W
    sl = pl.ds(base, B_PER_W)
    pltpu.sync_copy(uid.at[sl], uidx)
    pltpu.sync_copy(pid.at[sl], pidx)
    pltpu.sync_copy(nid.at[sl], nidx)

    lane = lax.iota(jnp.int32, LANES)
    one = jnp.full((LANES,), 1, jnp.int32)

    def lane_sum(v):
        for k in (8, 4, 2, 1):
            v = v + lax.gather(v, (lane ^ k)[:, None], _DNUMS, (1,),
                               mode=lax.GatherScatterMode.PROMISE_IN_BOUNDS)
        return v

    def bcast(v, j):
        return lax.gather(v, jnp.full((LANES, 1), j, jnp.int32), _DNUMS, (1,),
                          mode=lax.GatherScatterMode.PROMISE_IN_BOUNDS)

    for ch in range(NCHUNK):
        cbase = ch * CHUNK

        @pl.loop(0, CHUNK // LANES)
        def _(t):
            s16 = pl.ds(t * LANES, LANES)
            sg16 = pl.ds(cbase + t * LANES, LANES)
            g_u[s16] = lax.shift_right_logical(uidx[sg16], 1)
            g_p[s16] = lax.shift_right_logical(pidx[sg16], 1)
            g_n[s16] = lax.shift_right_logical(nidx[sg16], 1)

        cu = pltpu.async_copy(u_tab.at[g_u], urows, sem)
        cp = pltpu.async_copy(i_tab.at[g_p], prows, sem)
        cn = pltpu.async_copy(i_tab.at[g_n], nrows, sem)
        cu.wait()
        cp.wait()
        cn.wait()

        @pl.loop(0, CHUNK // LANES)
        def _(g):
            sg16 = pl.ds(cbase + g * LANES, LANES)
            upar = (uidx[sg16] & one).astype(jnp.float32)
            ppar = (pidx[sg16] & one).astype(jnp.float32)
            npar = (nidx[sg16] & one).astype(jnp.float32)
            accp = jnp.zeros((LANES,), jnp.float32)
            accn = jnp.zeros((LANES,), jnp.float32)
            for j in range(LANES):
                li = g * LANES + j
                mu = bcast(upar, j)
                mp = bcast(ppar, j)
                mn = bcast(npar, j)
                sp = jnp.zeros((LANES,), jnp.float32)
                sn = jnp.zeros((LANES,), jnp.float32)
                for c in range(EMBED_DIM // LANES):
                    ulo = urows[li, pl.ds(c * LANES, LANES)]
                    uhi = urows[li, pl.ds(EMBED_DIM + c * LANES, LANES)]
                    u = ulo + (uhi - ulo) * mu
                    plo = prows[li, pl.ds(c * LANES, LANES)]
                    phi = prows[li, pl.ds(EMBED_DIM + c * LANES, LANES)]
                    p = plo + (phi - plo) * mp
                    nlo = nrows[li, pl.ds(c * LANES, LANES)]
                    nhi = nrows[li, pl.ds(EMBED_DIM + c * LANES, LANES)]
                    n = nlo + (nhi - nlo) * mn
                    sp = sp + u * p
                    sn = sn + u * n
                mask = lane == j
                accp = jnp.where(mask, lane_sum(sp), accp)
                accn = jnp.where(mask, lane_sum(sn), accn)
            pos_v[sg16] = accp
            neg_v[sg16] = accn

    wp = pltpu.async_copy(pos_v, pos_out.at[sl], sem)
    wn = pltpu.async_copy(neg_v, neg_out.at[sl], sem)
    wp.wait()
    wn.wait()


def kernel(user_ids, pos_item_ids, neg_item_ids, user_table, item_table):
    uid = user_ids.astype(jnp.int32)
    pid = pos_item_ids.astype(jnp.int32)
    nid = neg_item_ids.astype(jnp.int32)
    u_tail = user_table[NFULL * CW:].reshape(TAILROWS // 2, TCOLS)
    i_tail = item_table[NFULL * CW:].reshape(TAILROWS // 2, TCOLS)
    ut2 = _k0(user_table.T, u_tail)
    it2 = _k0(item_table.T, i_tail)
    return tuple(_k1(ut2, it2, uid, pid, nid))


# speedup vs baseline: 2.9442x; 2.9442x over previous
import functools

import jax
import jax.numpy as jnp
from jax import lax
from jax.experimental import pallas as pl
from jax.experimental.pallas import tpu as pltpu
from jax.experimental.pallas import tpu_sc as plsc

NUM_CORES = 2
NUM_SUBCORES = 16
NUM_WORKERS = NUM_CORES * NUM_SUBCORES
BATCH = 16384
EMBED_DIM = 64
B_PER_W = BATCH // NUM_WORKERS
CHUNK = 256
NCHUNK = B_PER_W // CHUNK
LANES = 16
TROWS = 1000000
TCOLS = 128

_mesh = plsc.VectorSubcoreMesh(core_axis_name="c", subcore_axis_name="s")


@functools.partial(
    pl.kernel,
    mesh=_mesh,
    compiler_params=pltpu.CompilerParams(use_tc_tiling_on_sc=True),
    out_type=[
        jax.ShapeDtypeStruct((BATCH,), jnp.float32),
        jax.ShapeDtypeStruct((BATCH,), jnp.float32),
    ],
    scratch_types=[
        pltpu.VMEM((B_PER_W,), jnp.int32),
        pltpu.VMEM((B_PER_W,), jnp.int32),
        pltpu.VMEM((B_PER_W,), jnp.int32),
        pltpu.VMEM((CHUNK, TCOLS), jnp.float32),
        pltpu.VMEM((CHUNK, TCOLS), jnp.float32),
        pltpu.VMEM((CHUNK, TCOLS), jnp.float32),
        pltpu.VMEM((B_PER_W,), jnp.float32),
        pltpu.VMEM((B_PER_W,), jnp.float32),
        pltpu.SemaphoreType.DMA,
    ],
)
def _sc_fused(u_tab, i_tab, uid, pid, nid, pos_out, neg_out,
              uidx, pidx, nidx, urows, prows, nrows, pos_v, neg_v, sem):
    wid = lax.axis_index("s") * NUM_CORES + lax.axis_index("c")
    base = wid * B_PER_W
    sl = pl.ds(base, B_PER_W)
    pltpu.sync_copy(uid.at[sl], uidx)
    pltpu.sync_copy(pid.at[sl], pidx)
    pltpu.sync_copy(nid.at[sl], nidx)

    lane = lax.iota(jnp.int32, LANES)
    dnums = lax.GatherDimensionNumbers(
        offset_dims=(), collapsed_slice_dims=(0,), start_index_map=(0,))

    def lane_sum(v):
        for k in (8, 4, 2, 1):
            v = v + lax.gather(v, (lane ^ k)[:, None], dnums, (1,),
                               mode=lax.GatherScatterMode.PROMISE_IN_BOUNDS)
        return v

    for ch in range(NCHUNK):
        cbase = ch * CHUNK
        cu = pltpu.async_copy(u_tab.at[uidx.at[pl.ds(cbase, CHUNK)]], urows, sem)
        cp = pltpu.async_copy(i_tab.at[pidx.at[pl.ds(cbase, CHUNK)]], prows, sem)
        cn = pltpu.async_copy(i_tab.at[nidx.at[pl.ds(cbase, CHUNK)]], nrows, sem)
        cu.wait()
        cp.wait()
        cn.wait()

        @pl.loop(0, CHUNK // LANES)
        def _(g):
            sg16 = pl.ds(cbase + g * LANES, LANES)
            accp = jnp.zeros((LANES,), jnp.float32)
            accn = jnp.zeros((LANES,), jnp.float32)
            for j in range(LANES):
                li = g * LANES + j
                sp = jnp.zeros((LANES,), jnp.float32)
                sn = jnp.zeros((LANES,), jnp.float32)
                for c in range(EMBED_DIM // LANES):
                    u = urows[li, pl.ds(c * LANES, LANES)]
                    sp = sp + u * prows[li, pl.ds(c * LANES, LANES)]
                    sn = sn + u * nrows[li, pl.ds(c * LANES, LANES)]
                mask = lane == j
                accp = jnp.where(mask, lane_sum(sp), accp)
                accn = jnp.where(mask, lane_sum(sn), accn)
            pos_v[sg16] = accp
            neg_v[sg16] = accn

    wp = pltpu.async_copy(pos_v, pos_out.at[sl], sem)
    wn = pltpu.async_copy(neg_v, neg_out.at[sl], sem)
    wp.wait()
    wn.wait()


def kernel(user_ids, pos_item_ids, neg_item_ids, user_table, item_table):
    uid = user_ids.astype(jnp.int32)
    pid = pos_item_ids.astype(jnp.int32)
    nid = neg_item_ids.astype(jnp.int32)
    ut2 = jnp.pad(user_table, ((0, 0), (0, TCOLS - EMBED_DIM)))
    it2 = jnp.pad(item_table, ((0, 0), (0, TCOLS - EMBED_DIM)))
    return tuple(_sc_fused(ut2, it2, uid, pid, nid))
